# baseline (speedup 1.0000x reference)
import jax
import jax.numpy as jnp
from jax import lax
from jax.experimental import pallas as pl
from jax.experimental.pallas import tpu as pltpu
from jax.experimental.pallas import tpu_sc as plsc

B, P, L = 16384, 128, 100
NC, NS, LANES = 2, 16, 16
NW = NC * NS

BT = 8192
BS = 512
NBLK = BT // BS

BSC = B - BT
BPW = BSC // NW
GROUPS = BPW // LANES
UNROLL = 8
Q = 2


def _sc_body(logits_hbm, label_hbm, tbl_hbm, out_hbm, x_v, lbl_v, tbl_v, tot_v,
             sem0, sem1, sem_l, sem_t):
    c = lax.axis_index("c")
    s = lax.axis_index("s")
    wid = c * NS + s
    base = BT + wid * BPW
    chunk = BPW * P // Q

    sems = [sem0, sem1]
    cps = [
        pltpu.async_copy(
            logits_hbm.at[pl.ds(base * P + q * chunk, chunk)],
            x_v.at[pl.ds(q * chunk, chunk)], sems[q])
        for q in range(Q)
    ]
    cl = pltpu.async_copy(label_hbm.at[pl.ds(base, BPW)], lbl_v, sem_l)
    ct = pltpu.async_copy(tbl_hbm, tbl_v, sem_t)

    lane = lax.iota(jnp.int32, LANES)
    zero = jnp.zeros((LANES,), jnp.float32)

    def group_body(g, tot):
        lbl = lbl_v[pl.ds(g * LANES, LANES)]
        xb = g * (LANES * P) + lane * (P + 1)
        mb = lbl * P + lane

        def step(_, carry):
            a0, a1, a2, a3, ix, im = carry
            accs = [a0, a1, a2, a3]
            for u in range(UNROLL):
                xv = plsc.load_gather(x_v, [ix + u])
                mv = plsc.load_gather(tbl_v, [im + u])
                d = xv - mv
                accs[u % 4] = accs[u % 4] + d * d
            return (accs[0], accs[1], accs[2], accs[3],
                    ix + UNROLL, im + UNROLL)

        a0, a1, a2, a3, ix, im = lax.fori_loop(
            0, (P - LANES) // UNROLL, step, (zero, zero, zero, zero, xb, mb))

        xrow = g * (LANES * P) + lane * P
        for u in range(LANES):
            fu = (lane + (P - LANES) + u) & (P - 1)
            xv = plsc.load_gather(x_v, [xrow + fu])
            mv = plsc.load_gather(tbl_v, [lbl * P + fu])
            d = xv - mv
            accs = [a0, a1, a2, a3]
            accs[u % 4] = accs[u % 4] + d * d
            a0, a1, a2, a3 = accs

        ss = (a0 + a1) + (a2 + a3)

        xc = jnp.maximum(ss, jnp.float32(1e-30))
        yi = jnp.int32(0x5F3759DF) - lax.shift_right_logical(
            lax.bitcast_convert_type(xc, jnp.int32), 1)
        y = lax.bitcast_convert_type(yi, jnp.float32)
        for _ in range(3):
            y = y * (jnp.float32(1.5) - jnp.float32(0.5) * xc * y * y)
        return tot + xc * y

    cl.wait()
    ct.wait()
    tot = zero
    gq = GROUPS // Q
    for q in range(Q):
        cps[q].wait()
        tot = lax.fori_loop(q * gq, (q + 1) * gq, group_body, tot)
    tot_v[...] = tot
    pltpu.sync_copy(tot_v, out_hbm.at[wid])


def _tc_norm_body(lbl_ref, x_ref, mean_ref, o_ref):
    i = pl.program_id(0)
    lbl = lbl_ref[0, 0, :]
    oh = (lbl[:, None]
          == lax.broadcasted_iota(jnp.int32, (BS, P), 1)).astype(jnp.float32)
    g = jnp.dot(oh, mean_ref[...], preferred_element_type=jnp.float32)
    d = x_ref[...] - g
    ss = jnp.sum(d * d, axis=1, keepdims=True)
    nrm = jnp.sqrt(ss)

    @pl.when(i == 0)
    def _():
        o_ref[0, 0] = 0.0

    o_ref[0, 0] += jnp.sum(nrm)


def _tc_finish_body(x_ref, t_ref, o_ref):
    o_ref[0, 0] = (jnp.sum(x_ref[...]) + t_ref[0, 0]) * (1.0 / B)


@jax.jit
def kernel(logits, label, mean_expand):
    label = label.astype(jnp.int32)
    mean_pad = jnp.zeros((P, P), jnp.float32).at[:L].set(mean_expand)

    sc = pl.kernel(
        _sc_body,
        out_type=jax.ShapeDtypeStruct((NW, LANES), jnp.float32),
        mesh=plsc.VectorSubcoreMesh(core_axis_name="c", subcore_axis_name="s"),
        compiler_params=pltpu.CompilerParams(needs_layout_passes=False),
        scratch_types=[
            pltpu.VMEM((BPW * P,), jnp.float32),
            pltpu.VMEM((BPW,), jnp.int32),
            pltpu.VMEM((L * P,), jnp.float32),
            pltpu.VMEM((LANES,), jnp.float32),
            pltpu.SemaphoreType.DMA,
            pltpu.SemaphoreType.DMA,
            pltpu.SemaphoreType.DMA,
            pltpu.SemaphoreType.DMA,
        ],
    )
    sc_partials = sc(logits.reshape(B * P), label, mean_expand.reshape(L * P))

    tc_partial = pl.pallas_call(
        _tc_norm_body,
        grid=(NBLK,),
        in_specs=[
            pl.BlockSpec((1, 1, BS), lambda i: (i, 0, 0)),
            pl.BlockSpec((BS, P), lambda i: (i, 0)),
            pl.BlockSpec((P, P), lambda i: (0, 0)),
        ],
        out_specs=pl.BlockSpec(memory_space=pltpu.SMEM),
        out_shape=jax.ShapeDtypeStruct((1, 1), jnp.float32),
    )(label.reshape(B // BS, 1, BS), logits, mean_pad)

    loss = pl.pallas_call(
        _tc_finish_body,
        in_specs=[
            pl.BlockSpec((4, P), lambda: (0, 0)),
            pl.BlockSpec(memory_space=pltpu.SMEM),
        ],
        out_shape=jax.ShapeDtypeStruct((1, 1), jnp.float32),
        out_specs=pl.BlockSpec(memory_space=pltpu.SMEM),
    )(sc_partials.reshape(4, P), tc_partial)
    return loss[0, 0]


# speedup vs baseline: 1.1774x; 1.1774x over previous
import jax
import jax.numpy as jnp
from jax import lax
from jax.experimental import pallas as pl
from jax.experimental.pallas import tpu as pltpu
from jax.experimental.pallas import tpu_sc as plsc

B, P, L = 16384, 128, 100
NC, NS, LANES = 2, 16, 16
NW = NC * NS

BT = 8192
BS = 1024
NBLK = BT // BS

BSC = B - BT
BPW = BSC // NW
GROUPS = BPW // LANES
UNROLL = 8
Q = 2


def _sc_body(logits_hbm, label_hbm, tbl_hbm, out_hbm, x_v, lbl_v, tbl_v, tot_v,
             sem0, sem1, sem_l, sem_t):
    c = lax.axis_index("c")
    s = lax.axis_index("s")
    wid = c * NS + s
    base = BT + wid * BPW
    chunk = BPW * P // Q

    sems = [sem0, sem1]
    cps = [
        pltpu.async_copy(
            logits_hbm.at[pl.ds(base * P + q * chunk, chunk)],
            x_v.at[pl.ds(q * chunk, chunk)], sems[q])
        for q in range(Q)
    ]
    cl = pltpu.async_copy(label_hbm.at[pl.ds(base, BPW)], lbl_v, sem_l)
    ct = pltpu.async_copy(tbl_hbm, tbl_v, sem_t)

    lane = lax.iota(jnp.int32, LANES)
    zero = jnp.zeros((LANES,), jnp.float32)

    def group_body(g, tot):
        lbl = lbl_v[pl.ds(g * LANES, LANES)]
        xb = g * (LANES * P) + lane * (P + 1)
        mb = lbl * P + lane

        def step(_, carry):
            a0, a1, a2, a3, ix, im = carry
            accs = [a0, a1, a2, a3]
            for u in range(UNROLL):
                xv = plsc.load_gather(x_v, [ix + u])
                mv = plsc.load_gather(tbl_v, [im + u])
                d = xv - mv
                accs[u % 4] = accs[u % 4] + d * d
            return (accs[0], accs[1], accs[2], accs[3],
                    ix + UNROLL, im + UNROLL)

        a0, a1, a2, a3, ix, im = lax.fori_loop(
            0, (P - LANES) // UNROLL, step, (zero, zero, zero, zero, xb, mb))

        xrow = g * (LANES * P) + lane * P
        for u in range(LANES):
            fu = (lane + (P - LANES) + u) & (P - 1)
            xv = plsc.load_gather(x_v, [xrow + fu])
            mv = plsc.load_gather(tbl_v, [lbl * P + fu])
            d = xv - mv
            accs = [a0, a1, a2, a3]
            accs[u % 4] = accs[u % 4] + d * d
            a0, a1, a2, a3 = accs

        ss = (a0 + a1) + (a2 + a3)

        xc = jnp.maximum(ss, jnp.float32(1e-30))
        yi = jnp.int32(0x5F3759DF) - lax.shift_right_logical(
            lax.bitcast_convert_type(xc, jnp.int32), 1)
        y = lax.bitcast_convert_type(yi, jnp.float32)
        for _ in range(3):
            y = y * (jnp.float32(1.5) - jnp.float32(0.5) * xc * y * y)
        return tot + xc * y

    cl.wait()
    ct.wait()
    tot = zero
    gq = GROUPS // Q
    for q in range(Q):
        cps[q].wait()
        tot = lax.fori_loop(q * gq, (q + 1) * gq, group_body, tot)
    tot_v[...] = tot
    pltpu.sync_copy(tot_v, out_hbm.at[pl.ds(wid * LANES, LANES)])


def _tc_norm_body(lbl_ref, x_ref, mean_ref, o_ref):
    i = pl.program_id(0)
    lbl = lbl_ref[0, 0, :]
    oh = (lbl[:, None]
          == lax.broadcasted_iota(jnp.int32, (BS, P), 1)).astype(jnp.float32)
    g = jnp.dot(oh, mean_ref[...], preferred_element_type=jnp.float32)
    d = x_ref[...] - g
    ss = jnp.dot(d * d, jnp.ones((P, 1), jnp.float32),
                 preferred_element_type=jnp.float32)
    nrm = jnp.sqrt(ss)

    @pl.when(i == 0)
    def _():
        o_ref[0, 0] = 0.0

    o_ref[0, 0] += jnp.sum(nrm)


def _tc_finish_body(x_ref, t_ref, o_ref):
    o_ref[0, 0] = (jnp.sum(x_ref[...]) + t_ref[0, 0]) * (1.0 / B)


@jax.jit
def kernel(logits, label, mean_expand):
    label = label.astype(jnp.int32)
    mean_pad = jnp.zeros((P, P), jnp.float32).at[:L].set(mean_expand)

    sc = pl.kernel(
        _sc_body,
        out_type=jax.ShapeDtypeStruct((NW * LANES,), jnp.float32),
        mesh=plsc.VectorSubcoreMesh(core_axis_name="c", subcore_axis_name="s"),
        compiler_params=pltpu.CompilerParams(needs_layout_passes=False),
        scratch_types=[
            pltpu.VMEM((BPW * P,), jnp.float32),
            pltpu.VMEM((BPW,), jnp.int32),
            pltpu.VMEM((L * P,), jnp.float32),
            pltpu.VMEM((LANES,), jnp.float32),
            pltpu.SemaphoreType.DMA,
            pltpu.SemaphoreType.DMA,
            pltpu.SemaphoreType.DMA,
            pltpu.SemaphoreType.DMA,
        ],
    )
    sc_partials = sc(logits.reshape(B * P), label, mean_expand.reshape(L * P))

    tc_partial = pl.pallas_call(
        _tc_norm_body,
        grid=(NBLK,),
        in_specs=[
            pl.BlockSpec((1, 1, BS), lambda i: (i, 0, 0)),
            pl.BlockSpec((BS, P), lambda i: (i, 0)),
            pl.BlockSpec((P, P), lambda i: (0, 0)),
        ],
        out_specs=pl.BlockSpec(memory_space=pltpu.SMEM),
        out_shape=jax.ShapeDtypeStruct((1, 1), jnp.float32),
    )(label.reshape(B // BS, 1, BS), logits, mean_pad)

    loss = pl.pallas_call(
        _tc_finish_body,
        in_specs=[
            pl.BlockSpec((4, P), lambda: (0, 0)),
            pl.BlockSpec(memory_space=pltpu.SMEM),
        ],
        out_shape=jax.ShapeDtypeStruct((1, 1), jnp.float32),
        out_specs=pl.BlockSpec(memory_space=pltpu.SMEM),
    )(sc_partials.reshape(4, P), tc_partial)
    return loss[0, 0]


# speedup vs baseline: 1.1895x; 1.0103x over previous
import jax
import jax.numpy as jnp
from jax import lax
from jax.experimental import pallas as pl
from jax.experimental.pallas import tpu as pltpu
from jax.experimental.pallas import tpu_sc as plsc

B, P, L = 16384, 128, 100
NC, NS, LANES = 2, 16, 16
NW = NC * NS

BT = 8192
BS = 2048
NBLK = BT // BS
BROWS = BS // 128

BSC = B - BT
BPW = BSC // NW
GROUPS = BPW // LANES
UNROLL = 8
Q = 2


def _sc_body(logits_hbm, label_hbm, tbl_hbm, out_hbm, x_v, lbl_v, tbl_v, tot_v,
             sem0, sem1, sem_l, sem_t):
    c = lax.axis_index("c")
    s = lax.axis_index("s")
    wid = c * NS + s
    base = BT + wid * BPW
    chunk = BPW * P // Q

    sems = [sem0, sem1]
    cps = [
        pltpu.async_copy(
            logits_hbm.at[pl.ds(base * P + q * chunk, chunk)],
            x_v.at[pl.ds(q * chunk, chunk)], sems[q])
        for q in range(Q)
    ]
    cl = pltpu.async_copy(label_hbm.at[pl.ds(base, BPW)], lbl_v, sem_l)
    ct = pltpu.async_copy(tbl_hbm, tbl_v, sem_t)

    lane = lax.iota(jnp.int32, LANES)
    zero = jnp.zeros((LANES,), jnp.float32)

    def group_body(g, tot):
        lbl = lbl_v[pl.ds(g * LANES, LANES)]
        xb = g * (LANES * P) + lane * (P + 1)
        mb = lbl * P + lane

        @plsc.parallel_loop(0, (P - LANES) // UNROLL,
                            carry=((zero,) * UNROLL, xb, mb))
        def loop_a(_, carry):
            accs, ix, im = carry
            accs = list(accs)
            for u in range(UNROLL):
                xv = plsc.load_gather(x_v, [ix + u])
                mv = plsc.load_gather(tbl_v, [im + u])
                d = xv - mv
                accs[u] = accs[u] + d * d
            return tuple(accs), ix + UNROLL, im + UNROLL

        accs, ix, im = loop_a
        accs = list(accs)

        xrow = g * (LANES * P) + lane * P
        for u in range(LANES):
            fu = (lane + (P - LANES) + u) & (P - 1)
            xv = plsc.load_gather(x_v, [xrow + fu])
            mv = plsc.load_gather(tbl_v, [lbl * P + fu])
            d = xv - mv
            accs[u % UNROLL] = accs[u % UNROLL] + d * d

        s0 = (accs[0] + accs[1]) + (accs[2] + accs[3])
        s1 = (accs[4] + accs[5]) + (accs[6] + accs[7])
        ss = s0 + s1

        xc = jnp.maximum(ss, jnp.float32(1e-30))
        yi = jnp.int32(0x5F3759DF) - lax.shift_right_logical(
            lax.bitcast_convert_type(xc, jnp.int32), 1)
        y = lax.bitcast_convert_type(yi, jnp.float32)
        for _ in range(3):
            y = y * (jnp.float32(1.5) - jnp.float32(0.5) * xc * y * y)
        return tot + xc * y

    cl.wait()
    ct.wait()
    tot = zero
    gq = GROUPS // Q
    for q in range(Q):
        cps[q].wait()
        tot = lax.fori_loop(q * gq, (q + 1) * gq, group_body, tot)
    tot_v[...] = tot
    pltpu.sync_copy(tot_v, out_hbm.at[pl.ds(wid * LANES, LANES)])


def _tc_norm_body(lbl_ref, x_ref, mean_ref, o_ref):
    i = pl.program_id(0)
    lbl = lbl_ref[0, 0, :]
    oh = (lbl[:, None]
          == lax.broadcasted_iota(jnp.int32, (BS, L), 1)).astype(jnp.float32)
    g = jnp.dot(oh, mean_ref[...], preferred_element_type=jnp.float32)
    d = x_ref[...] - g
    ss = jnp.dot(d * d, jnp.ones((P, 1), jnp.float32),
                 preferred_element_type=jnp.float32)
    nrm = jnp.sqrt(ss)

    @pl.when(i == 0)
    def _():
        o_ref[0, 0] = 0.0

    o_ref[0, 0] += jnp.sum(nrm)


def _tc_finish_body(x_ref, t_ref, o_ref):
    o_ref[0, 0] = (jnp.sum(x_ref[...]) + t_ref[0, 0]) * (1.0 / B)


@jax.jit
def kernel(logits, label, mean_expand):
    label = label.astype(jnp.int32)

    sc = pl.kernel(
        _sc_body,
        out_type=jax.ShapeDtypeStruct((NW * LANES,), jnp.float32),
        mesh=plsc.VectorSubcoreMesh(core_axis_name="c", subcore_axis_name="s"),
        compiler_params=pltpu.CompilerParams(needs_layout_passes=False),
        scratch_types=[
            pltpu.VMEM((BPW * P,), jnp.float32),
            pltpu.VMEM((BPW,), jnp.int32),
            pltpu.VMEM((L * P,), jnp.float32),
            pltpu.VMEM((LANES,), jnp.float32),
            pltpu.SemaphoreType.DMA,
            pltpu.SemaphoreType.DMA,
            pltpu.SemaphoreType.DMA,
            pltpu.SemaphoreType.DMA,
        ],
    )
    sc_partials = sc(logits.reshape(B * P), label, mean_expand.reshape(L * P))

    tc_partial = pl.pallas_call(
        _tc_norm_body,
        grid=(NBLK,),
        in_specs=[
            pl.BlockSpec((1, 1, BS), lambda i: (i, 0, 0)),
            pl.BlockSpec((BS, P), lambda i: (i, 0)),
            pl.BlockSpec((L, P), lambda i: (0, 0)),
        ],
        out_specs=pl.BlockSpec(memory_space=pltpu.SMEM),
        out_shape=jax.ShapeDtypeStruct((1, 1), jnp.float32),
    )(label.reshape(B // BS, 1, BS), logits, mean_expand)

    loss = pl.pallas_call(
        _tc_finish_body,
        in_specs=[
            pl.BlockSpec((4, P), lambda: (0, 0)),
            pl.BlockSpec(memory_space=pltpu.SMEM),
        ],
        out_shape=jax.ShapeDtypeStruct((1, 1), jnp.float32),
        out_specs=pl.BlockSpec(memory_space=pltpu.SMEM),
    )(sc_partials.reshape(4, P), tc_partial)
    return loss[0, 0]


# speedup vs baseline: 1.2530x; 1.0533x over previous
import jax
import jax.numpy as jnp
from jax import lax
from jax.experimental import pallas as pl
from jax.experimental.pallas import tpu as pltpu
from jax.experimental.pallas import tpu_sc as plsc

B, P, L = 16384, 128, 100
NC, NS, LANES = 2, 16, 16
NW = NC * NS

BT = 12288
BS = 2048
NBLK = BT // BS
BROWS = BS // 128

BSC = B - BT
BPW = BSC // NW
GROUPS = BPW // LANES
UNROLL = 8
Q = 2


def _sc_body(logits_hbm, label_hbm, tbl_hbm, out_hbm, x_v, lbl_v, tbl_v, tot_v,
             sem0, sem1, sem_l, sem_t):
    c = lax.axis_index("c")
    s = lax.axis_index("s")
    wid = c * NS + s
    base = BT + wid * BPW
    chunk = BPW * P // Q

    sems = [sem0, sem1]
    cps = [
        pltpu.async_copy(
            logits_hbm.at[pl.ds(base * P + q * chunk, chunk)],
            x_v.at[pl.ds(q * chunk, chunk)], sems[q])
        for q in range(Q)
    ]
    cl = pltpu.async_copy(label_hbm.at[pl.ds(base, BPW)], lbl_v, sem_l)
    ct = pltpu.async_copy(tbl_hbm, tbl_v, sem_t)

    lane = lax.iota(jnp.int32, LANES)
    zero = jnp.zeros((LANES,), jnp.float32)

    def group_body(g, tot):
        lbl = lbl_v[pl.ds(g * LANES, LANES)]
        xb = g * (LANES * P) + lane * (P + 1)
        mb = lbl * P + lane

        @plsc.parallel_loop(0, (P - LANES) // UNROLL,
                            carry=((zero,) * UNROLL, xb, mb))
        def loop_a(_, carry):
            accs, ix, im = carry
            accs = list(accs)
            for u in range(UNROLL):
                xv = plsc.load_gather(x_v, [ix + u])
                mv = plsc.load_gather(tbl_v, [im + u])
                d = xv - mv
                accs[u] = accs[u] + d * d
            return tuple(accs), ix + UNROLL, im + UNROLL

        accs, ix, im = loop_a
        accs = list(accs)

        xrow = g * (LANES * P) + lane * P
        for u in range(LANES):
            fu = (lane + (P - LANES) + u) & (P - 1)
            xv = plsc.load_gather(x_v, [xrow + fu])
            mv = plsc.load_gather(tbl_v, [lbl * P + fu])
            d = xv - mv
            accs[u % UNROLL] = accs[u % UNROLL] + d * d

        s0 = (accs[0] + accs[1]) + (accs[2] + accs[3])
        s1 = (accs[4] + accs[5]) + (accs[6] + accs[7])
        ss = s0 + s1

        xc = jnp.maximum(ss, jnp.float32(1e-30))
        yi = jnp.int32(0x5F3759DF) - lax.shift_right_logical(
            lax.bitcast_convert_type(xc, jnp.int32), 1)
        y = lax.bitcast_convert_type(yi, jnp.float32)
        for _ in range(3):
            y = y * (jnp.float32(1.5) - jnp.float32(0.5) * xc * y * y)
        return tot + xc * y

    cl.wait()
    ct.wait()
    tot = zero
    gq = GROUPS // Q
    for q in range(Q):
        cps[q].wait()
        tot = lax.fori_loop(q * gq, (q + 1) * gq, group_body, tot)
    tot_v[...] = tot
    pltpu.sync_copy(tot_v, out_hbm.at[pl.ds(wid * LANES, LANES)])


def _tc_norm_body(lbl_ref, x_ref, mean_ref, o_ref):
    i = pl.program_id(0)
    lbl = lbl_ref[0, 0, :]
    oh = (lbl[:, None]
          == lax.broadcasted_iota(jnp.int32, (BS, L), 1)).astype(jnp.float32)
    g = jnp.dot(oh, mean_ref[...], preferred_element_type=jnp.float32)
    d = x_ref[...] - g
    ss = jnp.dot(d * d, jnp.ones((P, 1), jnp.float32),
                 preferred_element_type=jnp.float32)
    nrm = jnp.sqrt(ss)

    @pl.when(i == 0)
    def _():
        o_ref[0, 0] = 0.0

    o_ref[0, 0] += jnp.sum(nrm)


def _tc_finish_body(x_ref, t_ref, o_ref):
    o_ref[0, 0] = (jnp.sum(x_ref[...]) + t_ref[0, 0]) * (1.0 / B)


@jax.jit
def kernel(logits, label, mean_expand):
    label = label.astype(jnp.int32)

    sc = pl.kernel(
        _sc_body,
        out_type=jax.ShapeDtypeStruct((NW * LANES,), jnp.float32),
        mesh=plsc.VectorSubcoreMesh(core_axis_name="c", subcore_axis_name="s"),
        compiler_params=pltpu.CompilerParams(needs_layout_passes=False),
        scratch_types=[
            pltpu.VMEM((BPW * P,), jnp.float32),
            pltpu.VMEM((BPW,), jnp.int32),
            pltpu.VMEM((L * P,), jnp.float32),
            pltpu.VMEM((LANES,), jnp.float32),
            pltpu.SemaphoreType.DMA,
            pltpu.SemaphoreType.DMA,
            pltpu.SemaphoreType.DMA,
            pltpu.SemaphoreType.DMA,
        ],
    )
    sc_partials = sc(logits.reshape(B * P), label, mean_expand.reshape(L * P))

    tc_partial = pl.pallas_call(
        _tc_norm_body,
        grid=(NBLK,),
        in_specs=[
            pl.BlockSpec((1, 1, BS), lambda i: (i, 0, 0)),
            pl.BlockSpec((BS, P), lambda i: (i, 0)),
            pl.BlockSpec((L, P), lambda i: (0, 0)),
        ],
        out_specs=pl.BlockSpec(memory_space=pltpu.SMEM),
        out_shape=jax.ShapeDtypeStruct((1, 1), jnp.float32),
    )(label.reshape(B // BS, 1, BS), logits, mean_expand)

    loss = pl.pallas_call(
        _tc_finish_body,
        in_specs=[
            pl.BlockSpec((4, P), lambda: (0, 0)),
            pl.BlockSpec(memory_space=pltpu.SMEM),
        ],
        out_shape=jax.ShapeDtypeStruct((1, 1), jnp.float32),
        out_specs=pl.BlockSpec(memory_space=pltpu.SMEM),
    )(sc_partials.reshape(4, P), tc_partial)
    return loss[0, 0]


# speedup vs baseline: 1.2624x; 1.0075x over previous
import jax
import jax.numpy as jnp
from jax import lax
from jax.experimental import pallas as pl
from jax.experimental.pallas import tpu as pltpu
from jax.experimental.pallas import tpu_sc as plsc

B, P, L = 16384, 128, 100
NC, NS, LANES = 2, 16, 16
NW = NC * NS

BT = 12288
BS = 4096
NBLK = BT // BS
BROWS = BS // 128

BSC = B - BT
BPW = BSC // NW
GROUPS = BPW // LANES
UNROLL = 8
Q = 2


def _sc_body(logits_hbm, label_hbm, tbl_hbm, out_hbm, x_v, lbl_v, tbl_v, tot_v,
             sem0, sem1, sem_l, sem_t):
    c = lax.axis_index("c")
    s = lax.axis_index("s")
    wid = c * NS + s
    base = BT + wid * BPW
    chunk = BPW * P // Q

    sems = [sem0, sem1]
    cps = [
        pltpu.async_copy(
            logits_hbm.at[pl.ds(base * P + q * chunk, chunk)],
            x_v.at[pl.ds(q * chunk, chunk)], sems[q])
        for q in range(Q)
    ]
    cl = pltpu.async_copy(label_hbm.at[pl.ds(base, BPW)], lbl_v, sem_l)
    ct = pltpu.async_copy(tbl_hbm, tbl_v, sem_t)

    lane = lax.iota(jnp.int32, LANES)
    zero = jnp.zeros((LANES,), jnp.float32)

    def group_body(g, tot):
        lbl = lbl_v[pl.ds(g * LANES, LANES)]
        xb = g * (LANES * P) + lane * (P + 1)
        mb = lbl * P + lane

        @plsc.parallel_loop(0, (P - LANES) // UNROLL,
                            carry=((zero,) * UNROLL, xb, mb))
        def loop_a(_, carry):
            accs, ix, im = carry
            accs = list(accs)
            for u in range(UNROLL):
                xv = plsc.load_gather(x_v, [ix + u])
                mv = plsc.load_gather(tbl_v, [im + u])
                d = xv - mv
                accs[u] = accs[u] + d * d
            return tuple(accs), ix + UNROLL, im + UNROLL

        accs, ix, im = loop_a
        accs = list(accs)

        xrow = g * (LANES * P) + lane * P
        for u in range(LANES):
            fu = (lane + (P - LANES) + u) & (P - 1)
            xv = plsc.load_gather(x_v, [xrow + fu])
            mv = plsc.load_gather(tbl_v, [lbl * P + fu])
            d = xv - mv
            accs[u % UNROLL] = accs[u % UNROLL] + d * d

        s0 = (accs[0] + accs[1]) + (accs[2] + accs[3])
        s1 = (accs[4] + accs[5]) + (accs[6] + accs[7])
        ss = s0 + s1

        xc = jnp.maximum(ss, jnp.float32(1e-30))
        yi = jnp.int32(0x5F3759DF) - lax.shift_right_logical(
            lax.bitcast_convert_type(xc, jnp.int32), 1)
        y = lax.bitcast_convert_type(yi, jnp.float32)
        for _ in range(3):
            y = y * (jnp.float32(1.5) - jnp.float32(0.5) * xc * y * y)
        return tot + xc * y

    cl.wait()
    ct.wait()
    tot = zero
    gq = GROUPS // Q
    for q in range(Q):
        cps[q].wait()
        tot = lax.fori_loop(q * gq, (q + 1) * gq, group_body, tot)
    tot_v[...] = tot
    pltpu.sync_copy(tot_v, out_hbm.at[pl.ds(wid * LANES, LANES)])


def _tc_norm_body(lbl_ref, x_ref, mean_ref, o_ref):
    i = pl.program_id(0)
    lbl = lbl_ref[0, 0, :]
    oh = (lbl[:, None]
          == lax.broadcasted_iota(jnp.int32, (BS, L), 1)).astype(jnp.float32)
    g = jnp.dot(oh, mean_ref[...], preferred_element_type=jnp.float32)
    d = x_ref[...] - g
    ss = jnp.dot(d * d, jnp.ones((P, 1), jnp.float32),
                 preferred_element_type=jnp.float32)
    nrm = jnp.sqrt(ss)

    @pl.when(i == 0)
    def _():
        o_ref[0, 0] = 0.0

    o_ref[0, 0] += jnp.sum(nrm)


def _tc_finish_body(x_ref, t_ref, o_ref):
    o_ref[0, 0] = (jnp.sum(x_ref[...]) + t_ref[0, 0]) * (1.0 / B)


@jax.jit
def kernel(logits, label, mean_expand):
    label = label.astype(jnp.int32)

    sc = pl.kernel(
        _sc_body,
        out_type=jax.ShapeDtypeStruct((NW * LANES,), jnp.float32),
        mesh=plsc.VectorSubcoreMesh(core_axis_name="c", subcore_axis_name="s"),
        compiler_params=pltpu.CompilerParams(needs_layout_passes=False),
        scratch_types=[
            pltpu.VMEM((BPW * P,), jnp.float32),
            pltpu.VMEM((BPW,), jnp.int32),
            pltpu.VMEM((L * P,), jnp.float32),
            pltpu.VMEM((LANES,), jnp.float32),
            pltpu.SemaphoreType.DMA,
            pltpu.SemaphoreType.DMA,
            pltpu.SemaphoreType.DMA,
            pltpu.SemaphoreType.DMA,
        ],
    )
    sc_partials = sc(logits.reshape(B * P), label, mean_expand.reshape(L * P))

    tc_partial = pl.pallas_call(
        _tc_norm_body,
        grid=(NBLK,),
        in_specs=[
            pl.BlockSpec((1, 1, BS), lambda i: (i, 0, 0)),
            pl.BlockSpec((BS, P), lambda i: (i, 0)),
            pl.BlockSpec((L, P), lambda i: (0, 0)),
        ],
        out_specs=pl.BlockSpec(memory_space=pltpu.SMEM),
        out_shape=jax.ShapeDtypeStruct((1, 1), jnp.float32),
    )(label.reshape(B // BS, 1, BS), logits, mean_expand)

    loss = pl.pallas_call(
        _tc_finish_body,
        in_specs=[
            pl.BlockSpec((4, P), lambda: (0, 0)),
            pl.BlockSpec(memory_space=pltpu.SMEM),
        ],
        out_shape=jax.ShapeDtypeStruct((1, 1), jnp.float32),
        out_specs=pl.BlockSpec(memory_space=pltpu.SMEM),
    )(sc_partials.reshape(4, P), tc_partial)
    return loss[0, 0]


# speedup vs baseline: 1.3601x; 1.0774x over previous
import jax
import jax.numpy as jnp
from jax import lax
from jax.experimental import pallas as pl
from jax.experimental.pallas import tpu as pltpu
from jax.experimental.pallas import tpu_sc as plsc

B, P, L = 16384, 128, 100
NC, NS, LANES = 2, 16, 16
NW = NC * NS

BT = 12288
BS = 4096
NBLK = BT // BS
BROWS = BS // 128

BSC = B - BT
BPW = BSC // NW
GROUPS = BPW // LANES
UNROLL = 8
Q = 2


def _sc_body(logits_hbm, label_hbm, tbl_hbm, out_hbm, x_v, lbl_v, tbl_v, tot_v,
             tbl_spm, sem0, sem1, sem_l):
    c = lax.axis_index("c")
    s = lax.axis_index("s")
    wid = c * NS + s
    base = BT + wid * BPW
    chunk = BPW * P // Q

    sems = [sem0, sem1]
    cps = [
        pltpu.async_copy(
            logits_hbm.at[pl.ds(base * P + q * chunk, chunk)],
            x_v.at[pl.ds(q * chunk, chunk)], sems[q])
        for q in range(Q)
    ]
    cl = pltpu.async_copy(label_hbm.at[pl.ds(base, BPW)], lbl_v, sem_l)

    @pl.when(s == 0)
    def _():
        pltpu.sync_copy(tbl_hbm, tbl_spm)

    plsc.subcore_barrier()
    pltpu.sync_copy(tbl_spm, tbl_v)

    lane = lax.iota(jnp.int32, LANES)
    zero = jnp.zeros((LANES,), jnp.float32)

    def group_body(g, tot):
        lbl = lbl_v[pl.ds(g * LANES, LANES)]
        xb = g * (LANES * P) + lane * (P + 1)
        mb = lbl * P + lane

        @plsc.parallel_loop(0, (P - LANES) // UNROLL,
                            carry=((zero,) * UNROLL, xb, mb))
        def loop_a(_, carry):
            accs, ix, im = carry
            accs = list(accs)
            for u in range(UNROLL):
                xv = plsc.load_gather(x_v, [ix + u])
                mv = plsc.load_gather(tbl_v, [im + u])
                d = xv - mv
                accs[u] = accs[u] + d * d
            return tuple(accs), ix + UNROLL, im + UNROLL

        accs, ix, im = loop_a
        accs = list(accs)

        xrow = g * (LANES * P) + lane * P
        for u in range(LANES):
            fu = (lane + (P - LANES) + u) & (P - 1)
            xv = plsc.load_gather(x_v, [xrow + fu])
            mv = plsc.load_gather(tbl_v, [lbl * P + fu])
            d = xv - mv
            accs[u % UNROLL] = accs[u % UNROLL] + d * d

        s0 = (accs[0] + accs[1]) + (accs[2] + accs[3])
        s1 = (accs[4] + accs[5]) + (accs[6] + accs[7])
        ss = s0 + s1

        xc = jnp.maximum(ss, jnp.float32(1e-30))
        yi = jnp.int32(0x5F3759DF) - lax.shift_right_logical(
            lax.bitcast_convert_type(xc, jnp.int32), 1)
        y = lax.bitcast_convert_type(yi, jnp.float32)
        for _ in range(3):
            y = y * (jnp.float32(1.5) - jnp.float32(0.5) * xc * y * y)
        return tot + xc * y

    cl.wait()
    tot = zero
    gq = GROUPS // Q
    for q in range(Q):
        cps[q].wait()
        tot = lax.fori_loop(q * gq, (q + 1) * gq, group_body, tot)
    tot_v[...] = tot
    pltpu.sync_copy(tot_v, out_hbm.at[pl.ds(wid * LANES, LANES)])


def _tc_norm_body(lbl_ref, x_ref, mean_ref, o_ref):
    i = pl.program_id(0)
    lbl = lbl_ref[0, 0, :]
    oh = (lbl[:, None]
          == lax.broadcasted_iota(jnp.int32, (BS, L), 1)).astype(jnp.float32)
    g = jnp.dot(oh, mean_ref[...], preferred_element_type=jnp.float32)
    d = x_ref[...] - g
    ss = jnp.dot(d * d, jnp.ones((P, 1), jnp.float32),
                 preferred_element_type=jnp.float32)
    nrm = jnp.sqrt(ss)

    @pl.when(i == 0)
    def _():
        o_ref[0, 0] = 0.0

    o_ref[0, 0] += jnp.sum(nrm)


def _tc_finish_body(x_ref, t_ref, o_ref):
    o_ref[0, 0] = (jnp.sum(x_ref[...]) + t_ref[0, 0]) * (1.0 / B)


@jax.jit
def kernel(logits, label, mean_expand):
    label = label.astype(jnp.int32)

    sc = pl.kernel(
        _sc_body,
        out_type=jax.ShapeDtypeStruct((NW * LANES,), jnp.float32),
        mesh=plsc.VectorSubcoreMesh(core_axis_name="c", subcore_axis_name="s"),
        compiler_params=pltpu.CompilerParams(needs_layout_passes=False),
        scratch_types=[
            pltpu.VMEM((BPW * P,), jnp.float32),
            pltpu.VMEM((BPW,), jnp.int32),
            pltpu.VMEM((L * P,), jnp.float32),
            pltpu.VMEM((LANES,), jnp.float32),
            pltpu.VMEM_SHARED((L * P,), jnp.float32),
            pltpu.SemaphoreType.DMA,
            pltpu.SemaphoreType.DMA,
            pltpu.SemaphoreType.DMA,
        ],
    )
    sc_partials = sc(logits.reshape(B * P), label, mean_expand.reshape(L * P))

    tc_partial = pl.pallas_call(
        _tc_norm_body,
        grid=(NBLK,),
        in_specs=[
            pl.BlockSpec((1, 1, BS), lambda i: (i, 0, 0)),
            pl.BlockSpec((BS, P), lambda i: (i, 0)),
            pl.BlockSpec((L, P), lambda i: (0, 0)),
        ],
        out_specs=pl.BlockSpec(memory_space=pltpu.SMEM),
        out_shape=jax.ShapeDtypeStruct((1, 1), jnp.float32),
    )(label.reshape(B // BS, 1, BS), logits, mean_expand)

    loss = pl.pallas_call(
        _tc_finish_body,
        in_specs=[
            pl.BlockSpec((4, P), lambda: (0, 0)),
            pl.BlockSpec(memory_space=pltpu.SMEM),
        ],
        out_shape=jax.ShapeDtypeStruct((1, 1), jnp.float32),
        out_specs=pl.BlockSpec(memory_space=pltpu.SMEM),
    )(sc_partials.reshape(4, P), tc_partial)
    return loss[0, 0]
